# baseline (speedup 1.0000x reference)
import functools
import jax
import jax.numpy as jnp
from jax import lax
from jax.experimental import pallas as pl
from jax.experimental.pallas import tpu as pltpu
from jax.experimental.pallas import tpu_sc as plsc

D_M = 64
SCALE = 8.0
NW = 32
CHUNK = 128
LANES = 16


def _emb_call(B):
    J = B // (NW * CHUNK)
    mesh = plsc.VectorSubcoreMesh(core_axis_name="c", subcore_axis_name="s")

    @functools.partial(
        pl.kernel,
        mesh=mesh,
        out_type=jax.ShapeDtypeStruct((B, D_M), jnp.float32),
        compiler_params=pltpu.CompilerParams(use_tc_tiling_on_sc=False),
        scratch_types=[
            pltpu.VMEM((J, CHUNK), jnp.int32),
            pltpu.VMEM((CHUNK, D_M), jnp.float32),
            pltpu.SemaphoreType.DMA,
        ],
    )
    def body(idx_hbm, lut_hbm, out_hbm, idx_v, buf_v, gsem):
        wid = lax.axis_index("s") * 2 + lax.axis_index("c")
        rbase = wid * J
        pltpu.sync_copy(idx_hbm.at[pl.ds(rbase, J)], idx_v)

        def step(j, carry):
            pltpu.async_copy(lut_hbm.at[idx_v.at[j]], buf_v, gsem).wait()

            def srow(r, c2):
                for q in range(D_M // LANES):
                    sl = pl.ds(q * LANES, LANES)
                    buf_v[r, sl] = buf_v[r, sl] * SCALE
                return c2

            lax.fori_loop(0, CHUNK, srow, 0)
            pltpu.sync_copy(buf_v, out_hbm.at[pl.ds((rbase + j) * CHUNK, CHUNK)])
            return carry

        lax.fori_loop(0, J, step, 0)

    return body


def kernel(x, lut):
    B = x.shape[0] * x.shape[1]
    xf = x.astype(jnp.int32).reshape(B // CHUNK, CHUNK)
    out = _emb_call(B)(xf, lut)
    return out.reshape(x.shape[0], x.shape[1], D_M)


# speedup vs baseline: 1.2089x; 1.2089x over previous
import functools
import jax
import jax.numpy as jnp
from jax import lax
from jax.experimental import pallas as pl
from jax.experimental.pallas import tpu as pltpu
from jax.experimental.pallas import tpu_sc as plsc

D_M = 64
SCALE = 8.0
NW = 32
CHUNK = 128
LANES = 16
NBUF = 8
AHEAD = 4


def _emb_call(B):
    J = B // (NW * CHUNK)
    mesh = plsc.VectorSubcoreMesh(core_axis_name="c", subcore_axis_name="s")

    @functools.partial(
        pl.kernel,
        mesh=mesh,
        out_type=jax.ShapeDtypeStruct((B, D_M), jnp.float32),
        compiler_params=pltpu.CompilerParams(use_tc_tiling_on_sc=False),
        scratch_types=[
            pltpu.VMEM((J, CHUNK), jnp.int32),
            pltpu.VMEM((NBUF, CHUNK, D_M), jnp.float32),
            pltpu.SemaphoreType.DMA((NBUF,)),
            pltpu.SemaphoreType.DMA((NBUF,)),
        ],
    )
    def body(idx_hbm, lut_hbm, out_hbm, idx_v, bufs, gsems, osems):
        wid = lax.axis_index("s") * 2 + lax.axis_index("c")
        rbase = wid * J
        pltpu.sync_copy(idx_hbm.at[pl.ds(rbase, J)], idx_v)

        for b in range(AHEAD):
            pltpu.async_copy(lut_hbm.at[idx_v.at[b]], bufs.at[b], gsems.at[b])

        def block(j0, carry):
            for b in range(NBUF):
                j = j0 + b
                jf = j + AHEAD
                bf = (b + AHEAD) % NBUF

                @pl.when(jf < J)
                def _fire():
                    @pl.when(jf >= NBUF)
                    def _drain():
                        pltpu.make_async_copy(
                            bufs.at[bf],
                            out_hbm.at[pl.ds((rbase + jf - NBUF) * CHUNK, CHUNK)],
                            osems.at[bf],
                        ).wait()

                    pltpu.async_copy(
                        lut_hbm.at[idx_v.at[jf]], bufs.at[bf], gsems.at[bf]
                    )

                pltpu.make_async_copy(
                    lut_hbm.at[idx_v.at[j]], bufs.at[b], gsems.at[b]
                ).wait()

                def srow(r, c2):
                    for rr in range(2):
                        for q in range(D_M // LANES):
                            sl = pl.ds(q * LANES, LANES)
                            bufs[b, 2 * r + rr, sl] = bufs[b, 2 * r + rr, sl] * SCALE
                    return c2

                lax.fori_loop(0, CHUNK // 2, srow, 0, unroll=2)

                pltpu.async_copy(
                    bufs.at[b],
                    out_hbm.at[pl.ds((rbase + j) * CHUNK, CHUNK)],
                    osems.at[b],
                )
            return carry

        lax.fori_loop(0, J // NBUF, lambda t, c: block(t * NBUF, c), 0)

        for b in range(NBUF):
            j_last = J - NBUF + b
            pltpu.make_async_copy(
                bufs.at[b],
                out_hbm.at[pl.ds((rbase + j_last) * CHUNK, CHUNK)],
                osems.at[b],
            ).wait()

    return body


def kernel(x, lut):
    B = x.shape[0] * x.shape[1]
    xf = x.astype(jnp.int32).reshape(B // CHUNK, CHUNK)
    out = _emb_call(B)(xf, lut)
    return out.reshape(x.shape[0], x.shape[1], D_M)
